# Initial kernel scaffold; baseline (speedup 1.0000x reference)
import jax
import jax.numpy as jnp
from jax.experimental import pallas as pl


def kernel(indices, weight):
    raise NotImplementedError("write your pallas kernel here")


# speedup vs baseline: 4.8011x; 4.8011x over previous
import functools

import jax
import jax.numpy as jnp
from jax import lax
from jax.experimental import pallas as pl
from jax.experimental.pallas import tpu as pltpu
from jax.experimental.pallas import tpu_sc as plsc

NUM_CORES = 2
NUM_SUBCORES = 16
NW = NUM_CORES * NUM_SUBCORES

CHUNK = 1024


@functools.partial(jax.jit, static_argnames=("b_per_w", "n_chunks"))
def _sc_gather(weight, flat_idx, *, b_per_w, n_chunks):
    B = flat_idx.shape[0]
    D = weight.shape[1]
    mesh = plsc.VectorSubcoreMesh(
        core_axis_name="c", subcore_axis_name="s",
        num_cores=NUM_CORES, num_subcores=NUM_SUBCORES,
    )

    @functools.partial(
        pl.kernel,
        out_type=jax.ShapeDtypeStruct((B, D), jnp.float32),
        mesh=mesh,
        scratch_types=[
            pltpu.VMEM((CHUNK,), jnp.int32),
            pltpu.VMEM((CHUNK, D), jnp.float32),
            pltpu.SemaphoreType.DMA,
        ],
        compiler_params=pltpu.CompilerParams(use_tc_tiling_on_sc=False),
    )
    def k(table_hbm, idx_hbm, out_hbm, idx_v, rows_v, sem):
        wid = lax.axis_index("s") * NUM_CORES + lax.axis_index("c")
        w_base = wid * b_per_w

        @pl.loop(0, n_chunks)
        def _chunk(c):
            base = w_base + c * CHUNK
            pltpu.sync_copy(idx_hbm.at[pl.ds(base, CHUNK)], idx_v)
            pltpu.async_copy(table_hbm.at[idx_v], rows_v, sem).wait()
            pltpu.sync_copy(rows_v, out_hbm.at[pl.ds(base, CHUNK)])

    return k(weight, flat_idx)


def kernel(indices, weight):
    B = indices.size
    flat = indices.reshape(B).astype(jnp.int32)
    assert B % NW == 0
    b_per_w = B // NW
    assert b_per_w % CHUNK == 0
    out = _sc_gather(weight, flat, b_per_w=b_per_w, n_chunks=b_per_w // CHUNK)
    return out.reshape(*indices.shape, weight.shape[1])


# speedup vs baseline: 5.0438x; 1.0506x over previous
import functools

import jax
import jax.numpy as jnp
from jax import lax
from jax.experimental import pallas as pl
from jax.experimental.pallas import tpu as pltpu
from jax.experimental.pallas import tpu_sc as plsc

NUM_CORES = 2
NUM_SUBCORES = 16
NW = NUM_CORES * NUM_SUBCORES

CHUNK = 512
NBUF = 4


@functools.partial(jax.jit, static_argnames=("b_per_w", "n_chunks"))
def _sc_gather(weight, flat_idx, *, b_per_w, n_chunks):
    B = flat_idx.shape[0]
    D = weight.shape[1]
    mesh = plsc.VectorSubcoreMesh(
        core_axis_name="c", subcore_axis_name="s",
        num_cores=NUM_CORES, num_subcores=NUM_SUBCORES,
    )

    @functools.partial(
        pl.kernel,
        out_type=jax.ShapeDtypeStruct((B, D), jnp.float32),
        mesh=mesh,
        scratch_types=[
            pltpu.VMEM((NBUF, CHUNK), jnp.int32),
            pltpu.VMEM((NBUF, CHUNK, D), jnp.float32),
            [pltpu.SemaphoreType.DMA] * NBUF,
            [pltpu.SemaphoreType.DMA] * NBUF,
        ],
        compiler_params=pltpu.CompilerParams(use_tc_tiling_on_sc=False),
    )
    def k(table_hbm, idx_hbm, out_hbm, idx_v, rows_v, gsems, osems):
        wid = lax.axis_index("s") * NUM_CORES + lax.axis_index("c")
        w_base = wid * b_per_w

        def load_idx(c, b):
            pltpu.sync_copy(idx_hbm.at[pl.ds(w_base + c * CHUNK, CHUNK)],
                            idx_v.at[b])

        def gather(b):
            return pltpu.make_async_copy(table_hbm.at[idx_v.at[b]],
                                         rows_v.at[b], gsems[b])

        def store(c, b):
            return pltpu.make_async_copy(
                rows_v.at[b],
                out_hbm.at[pl.ds(w_base + c * CHUNK, CHUNK)], osems[b])

        for b in range(NBUF):
            load_idx(b, b)
            gather(b).start()

        @pl.loop(0, n_chunks, step=NBUF)
        def _ring(c0):
            for b in range(NBUF):
                c = c0 + b
                gather(b).wait()
                store(c, b).start()

                @pl.when(c + NBUF < n_chunks)
                def _refill():
                    load_idx(c + NBUF, b)
                    store(c, b).wait()
                    gather(b).start()

        for b in range(NBUF):
            store(n_chunks - NBUF + b, b).wait()

    return k(weight, flat_idx)


def kernel(indices, weight):
    B = indices.size
    flat = indices.reshape(B).astype(jnp.int32)
    assert B % NW == 0
    b_per_w = B // NW
    assert b_per_w % CHUNK == 0 and (b_per_w // CHUNK) % NBUF == 0
    out = _sc_gather(weight, flat, b_per_w=b_per_w, n_chunks=b_per_w // CHUNK)
    return out.reshape(*indices.shape, weight.shape[1])
